# baseline (speedup 1.0000x reference)
import functools

import jax
import jax.numpy as jnp
from jax.experimental import pallas as pl
from jax.experimental.pallas import tpu as pltpu

N_USERS = 16384
N_ITEMS = 2048
RANK = 128
BU = 512
NA = N_USERS // BU


def _bf16_to_f16_bits(x):
    t = jax.lax.bitcast_convert_type(x * jnp.bfloat16(2.0 ** -112), jnp.int16)
    mag = t & jnp.int16(0x7FFF)
    mag = mag + mag
    mag = mag + mag
    mag = mag + mag
    return (t & jnp.int16(-0x8000)) | mag


def _fused_kernel(lam_ref, isv_ref, usv_ref, nadj_ref, out_ref,
                  isv2_ref, bacc_ref, b16_ref, c_ref):
    i = pl.program_id(0)

    @pl.when(i == 0)
    def _prep():
        isv2_ref[...] = (isv_ref[...] * (1.0 / lam_ref[...])).astype(
            jnp.bfloat16)

    @pl.when(i < NA)
    def _phase_a():
        nblk = nadj_ref[...]
        nb16 = nblk.astype(jnp.bfloat16)
        adj_blk = jnp.ceil(nb16)
        usv16 = usv_ref[...].astype(jnp.bfloat16)
        part = jax.lax.dot_general(
            usv16, adj_blk, (((0,), (0,)), ((), ())),
            preferred_element_type=jnp.float32)

        @pl.when(i == 0)
        def _init():
            bacc_ref[...] = part

        @pl.when(i > 0)
        def _acc():
            bacc_ref[...] += part

        c_ref[pl.ds(i * BU, BU), :] = jax.lax.dot_general(
            nb16, isv2_ref[...], (((1,), (0,)), ((), ())),
            preferred_element_type=jnp.float32)

    @pl.when(i == NA)
    def _seal_b():
        b16_ref[...] = bacc_ref[...].astype(jnp.bfloat16)

    @pl.when(i >= NA)
    def _phase_b():
        j = i - NA
        c = c_ref[pl.ds(j * BU, BU), :].astype(jnp.bfloat16)
        r = jax.lax.dot_general(
            c, b16_ref[...], (((1,), (0,)), ((), ())),
            preferred_element_type=jnp.float32)
        out_ref.bitcast(jnp.int16)[...] = _bf16_to_f16_bits(
            r.astype(jnp.bfloat16))


@functools.partial(jax.jit, static_argnames=("interpret",))
def kernel(lambda_mat, adj_mat, norm_adj, user_sv, item_sv, interpret=False):
    del adj_mat
    lam_row = lambda_mat.reshape(1, RANK)
    rating = pl.pallas_call(
        _fused_kernel,
        grid=(2 * NA,),
        in_specs=[
            pl.BlockSpec((1, RANK), lambda i: (0, 0)),
            pl.BlockSpec((N_ITEMS, RANK), lambda i: (0, 0)),
            pl.BlockSpec((BU, RANK), lambda i: (jnp.minimum(i, NA - 1), 0)),
            pl.BlockSpec((BU, N_ITEMS), lambda i: (jnp.minimum(i, NA - 1), 0)),
        ],
        out_specs=pl.BlockSpec(
            (BU, N_ITEMS), lambda i: (jnp.maximum(i - NA, 0), 0)),
        out_shape=jax.ShapeDtypeStruct((N_USERS, N_ITEMS), jnp.float16),
        scratch_shapes=[
            pltpu.VMEM((N_ITEMS, RANK), jnp.bfloat16),
            pltpu.VMEM((RANK, N_ITEMS), jnp.float32),
            pltpu.VMEM((RANK, N_ITEMS), jnp.bfloat16),
            pltpu.VMEM((N_USERS, RANK), jnp.float32),
        ],
        interpret=interpret,
    )(lam_row, item_sv, user_sv, norm_adj)

    return rating


# speedup vs baseline: 1.2320x; 1.2320x over previous
import functools

import jax
import jax.numpy as jnp
from jax.experimental import pallas as pl
from jax.experimental.pallas import tpu as pltpu

N_USERS = 16384
N_ITEMS = 2048
RANK = 128
BU = 1024
NA = N_USERS // BU


def _bf16_to_f16_bits(x):
    t = jax.lax.bitcast_convert_type(x * jnp.bfloat16(2.0 ** -112), jnp.int16)
    mag = t & jnp.int16(0x7FFF)
    mag = mag + mag
    mag = mag + mag
    mag = mag + mag
    return (t & jnp.int16(-0x8000)) | mag


def _fused_kernel(lam_ref, isv_ref, usv_ref, nadj_ref, out_ref,
                  isv2_ref, bacc_ref, b16_ref, c_ref):
    i = pl.program_id(0)

    @pl.when(i == 0)
    def _prep():
        isv2_ref[...] = (isv_ref[...] * (1.0 / lam_ref[...])).astype(
            jnp.bfloat16)

    @pl.when(i < NA)
    def _phase_a():
        nblk = nadj_ref[...]
        nb16 = nblk.astype(jnp.bfloat16)
        adj_blk = jnp.ceil(nb16)
        usv16 = usv_ref[...].astype(jnp.bfloat16)
        part = jax.lax.dot_general(
            usv16, adj_blk, (((0,), (0,)), ((), ())),
            preferred_element_type=jnp.float32)

        @pl.when(i == 0)
        def _init():
            bacc_ref[...] = part

        @pl.when(i > 0)
        def _acc():
            bacc_ref[...] += part

        c_ref[pl.ds(i * BU, BU), :] = jax.lax.dot_general(
            nb16, isv2_ref[...], (((1,), (0,)), ((), ())),
            preferred_element_type=jnp.float32).astype(jnp.bfloat16)

    @pl.when(i == NA)
    def _seal_b():
        b16_ref[...] = bacc_ref[...].astype(jnp.bfloat16)

    @pl.when(i >= NA)
    def _phase_b():
        j = i - NA
        c = c_ref[pl.ds(j * BU, BU), :]
        r = jax.lax.dot_general(
            c, b16_ref[...], (((1,), (0,)), ((), ())),
            preferred_element_type=jnp.float32)
        out_ref.bitcast(jnp.int16)[...] = _bf16_to_f16_bits(
            r.astype(jnp.bfloat16))


@functools.partial(jax.jit, static_argnames=("interpret",))
def kernel(lambda_mat, adj_mat, norm_adj, user_sv, item_sv, interpret=False):
    del adj_mat
    lam_row = lambda_mat.reshape(1, RANK)
    rating = pl.pallas_call(
        _fused_kernel,
        grid=(2 * NA,),
        in_specs=[
            pl.BlockSpec((1, RANK), lambda i: (0, 0)),
            pl.BlockSpec((N_ITEMS, RANK), lambda i: (0, 0)),
            pl.BlockSpec((BU, RANK), lambda i: (jnp.minimum(i, NA - 1), 0)),
            pl.BlockSpec((BU, N_ITEMS), lambda i: (jnp.minimum(i, NA - 1), 0)),
        ],
        out_specs=pl.BlockSpec(
            (BU, N_ITEMS), lambda i: (jnp.maximum(i - NA, 0), 0)),
        out_shape=jax.ShapeDtypeStruct((N_USERS, N_ITEMS), jnp.float16),
        scratch_shapes=[
            pltpu.VMEM((N_ITEMS, RANK), jnp.bfloat16),
            pltpu.VMEM((RANK, N_ITEMS), jnp.float32),
            pltpu.VMEM((RANK, N_ITEMS), jnp.bfloat16),
            pltpu.VMEM((N_USERS, RANK), jnp.bfloat16),
        ],
        interpret=interpret,
    )(lam_row, item_sv, user_sv, norm_adj)

    return rating


# speedup vs baseline: 1.2617x; 1.0241x over previous
import functools

import jax
import jax.numpy as jnp
from jax.experimental import pallas as pl
from jax.experimental.pallas import tpu as pltpu

N_USERS = 16384
N_ITEMS = 2048
RANK = 128
BU = 1024
NA = N_USERS // BU
BUB = 2048
NB = N_USERS // BUB


def _bf16_to_f16_bits(x):
    t = jax.lax.bitcast_convert_type(x * jnp.bfloat16(2.0 ** -112), jnp.int16)
    mag = t & jnp.int16(0x7FFF)
    mag = mag + mag
    mag = mag + mag
    mag = mag + mag
    return (t & jnp.int16(-0x8000)) | mag


def _fused_kernel(lam_ref, isv_ref, usv_ref, nadj_ref, out_ref,
                  isv2_ref, bacc_ref, b16_ref, c_ref):
    i = pl.program_id(0)

    @pl.when(i == 0)
    def _prep():
        isv2_ref[...] = (isv_ref[...] * (1.0 / lam_ref[...])).astype(
            jnp.bfloat16)

    @pl.when(i < NA)
    def _phase_a():
        nblk = nadj_ref[...]
        nb16 = nblk.astype(jnp.bfloat16)
        adj_blk = jnp.ceil(nb16)
        usv16 = usv_ref[...].astype(jnp.bfloat16)
        part = jax.lax.dot_general(
            usv16, adj_blk, (((0,), (0,)), ((), ())),
            preferred_element_type=jnp.float32)

        @pl.when(i == 0)
        def _init():
            bacc_ref[...] = part

        @pl.when(i > 0)
        def _acc():
            bacc_ref[...] += part

        c_ref[pl.ds(i * BU, BU), :] = jax.lax.dot_general(
            nb16, isv2_ref[...], (((1,), (0,)), ((), ())),
            preferred_element_type=jnp.float32).astype(jnp.bfloat16)

    @pl.when(i == NA)
    def _seal_b():
        b16_ref[...] = bacc_ref[...].astype(jnp.bfloat16)

    @pl.when(i >= NA)
    def _phase_b():
        j = i - NA
        c = c_ref[pl.ds(j * BUB, BUB), :]
        r = jax.lax.dot_general(
            c, b16_ref[...], (((1,), (0,)), ((), ())),
            preferred_element_type=jnp.float32)
        out_ref.bitcast(jnp.int16)[...] = _bf16_to_f16_bits(
            r.astype(jnp.bfloat16))


@functools.partial(jax.jit, static_argnames=("interpret",))
def kernel(lambda_mat, adj_mat, norm_adj, user_sv, item_sv, interpret=False):
    del adj_mat
    lam_row = lambda_mat.reshape(1, RANK)
    rating = pl.pallas_call(
        _fused_kernel,
        grid=(NA + NB,),
        in_specs=[
            pl.BlockSpec((1, RANK), lambda i: (0, 0)),
            pl.BlockSpec((N_ITEMS, RANK), lambda i: (0, 0)),
            pl.BlockSpec((BU, RANK), lambda i: (jnp.minimum(i, NA - 1), 0)),
            pl.BlockSpec((BU, N_ITEMS), lambda i: (jnp.minimum(i, NA - 1), 0)),
        ],
        out_specs=pl.BlockSpec(
            (BUB, N_ITEMS), lambda i: (jnp.maximum(i - NA, 0), 0)),
        out_shape=jax.ShapeDtypeStruct((N_USERS, N_ITEMS), jnp.float16),
        scratch_shapes=[
            pltpu.VMEM((N_ITEMS, RANK), jnp.bfloat16),
            pltpu.VMEM((RANK, N_ITEMS), jnp.float32),
            pltpu.VMEM((RANK, N_ITEMS), jnp.bfloat16),
            pltpu.VMEM((N_USERS, RANK), jnp.bfloat16),
        ],
        interpret=interpret,
    )(lam_row, item_sv, user_sv, norm_adj)

    return rating
